# baseline (speedup 1.0000x reference)
import functools

import jax
import jax.numpy as jnp
from jax import lax
from jax.experimental import pallas as pl
from jax.experimental.pallas import tpu as pltpu
from jax.experimental.pallas import tpu_sc as plsc

_N_E = 1024
_E_DIM = 256
_BETA = 0.25
_B = 9216

_BB = 3072
_NSUB = 3
_SB = _BB // _NSUB
_NCHUNK = 1
_CB = _B // _NCHUNK
_NB = _CB // _BB

_NC = 2
_NS = 16
_NW = _NC * _NS


def _dist_body(z_ref, w_ref, z2_ref, w2_ref, idx_ref, loss_ref):
    wx = w_ref[...] + w_ref[...]
    w2b = w2_ref[...]
    part = jnp.zeros((1, 1), jnp.float32)
    for h in range(_NSUB):
        rows = pl.ds(h * _SB, _SB)
        zb = z_ref[rows, :]
        s2 = lax.dot_general(zb, wx, (((1,), (1,)), ((), ())))
        d2 = (z2_ref[rows, :] + w2b) - s2
        dists = jnp.sqrt(jnp.maximum(d2, 0.0))
        m = jnp.min(dists, axis=1, keepdims=True)
        colf = lax.broadcasted_iota(jnp.int32, dists.shape, 1).astype(jnp.float32)
        idx = jnp.min(jnp.where(dists == m, colf, 2048.0), axis=1)
        idx_ref[pl.ds(h * _SB, _SB)] = idx.astype(jnp.int32)
        part += jnp.sum(m * m, keepdims=True)
    @pl.when(pl.program_id(0) == 0)
    def _init():
        loss_ref[...] = jnp.zeros_like(loss_ref)
    loss_ref[...] += (1.0 + _BETA) * part


def _dist_call(z, W, z2, w2):
    return pl.pallas_call(
        _dist_body,
        grid=(_NB,),
        in_specs=[
            pl.BlockSpec((_BB, _E_DIM), lambda i: (i, 0)),
            pl.BlockSpec((_N_E, _E_DIM), lambda i: (0, 0)),
            pl.BlockSpec((_BB, 1), lambda i: (i, 0)),
            pl.BlockSpec((1, _N_E), lambda i: (0, 0)),
        ],
        out_specs=[
            pl.BlockSpec((_BB,), lambda i: (i,)),
            pl.BlockSpec((1, 1), lambda i: (0, 0)),
        ],
        out_shape=[
            jax.ShapeDtypeStruct((_CB,), jnp.int32),
            jax.ShapeDtypeStruct((1, 1), jnp.float32),
        ],
    )(z, W, z2, w2)


@functools.lru_cache(maxsize=1)
def _make_sc_gather():
    rows = _CB
    bpw = rows // _NW
    mesh = plsc.VectorSubcoreMesh(core_axis_name="c", subcore_axis_name="s")

    ngrp = 3
    gb = bpw // ngrp

    @functools.partial(
        pl.kernel,
        mesh=mesh,
        out_type=jax.ShapeDtypeStruct((rows, _E_DIM), jnp.float32),
        scratch_types=[
            pltpu.VMEM((bpw,), jnp.int32),
            pltpu.VMEM((bpw, _E_DIM), jnp.float32),
        ] + [pltpu.SemaphoreType.DMA] * ngrp,
    )
    def _sc_gather(w_hbm, idx_hbm, out_hbm, idx_v, rows_v, *sems):
        wid = lax.axis_index("s") * _NC + lax.axis_index("c")
        base = wid * bpw
        pltpu.sync_copy(idx_hbm.at[pl.ds(base, bpw)], idx_v)
        cps = [
            pltpu.async_copy(
                w_hbm.at[idx_v.at[pl.ds(g * gb, gb)]],
                rows_v.at[pl.ds(g * gb, gb)], sems[g])
            for g in range(ngrp)
        ]
        for g in range(ngrp):
            cps[g].wait()
            pltpu.sync_copy(rows_v.at[pl.ds(g * gb, gb)],
                            out_hbm.at[pl.ds(base + g * gb, gb)])

    return _sc_gather


def kernel(z, W):
    z2 = jnp.sum(z * z, axis=1, keepdims=True)
    w2 = jnp.sum(W * W, axis=1)[None, :]
    gather = _make_sc_gather()
    zq_chunks = []
    loss = None
    for c in range(_NCHUNK):
        zc = lax.slice_in_dim(z, c * _CB, (c + 1) * _CB, axis=0)
        z2c = lax.slice_in_dim(z2, c * _CB, (c + 1) * _CB, axis=0)
        idx, lc = _dist_call(zc, W, z2c, w2)
        zq_chunks.append(gather(W, idx))
        loss = lc if loss is None else loss + lc
    z_q = zq_chunks[0] if _NCHUNK == 1 else jnp.concatenate(zq_chunks, axis=0)
    return (loss[0, 0], z_q)


# speedup vs baseline: 1.0159x; 1.0159x over previous
import functools

import jax
import jax.numpy as jnp
from jax import lax
from jax.experimental import pallas as pl
from jax.experimental.pallas import tpu as pltpu
from jax.experimental.pallas import tpu_sc as plsc

_N_E = 1024
_E_DIM = 256
_BETA = 0.25
_B = 9216

_BB = 3072
_NSUB = 3
_SB = _BB // _NSUB
_NCHUNK = 1
_CB = _B // _NCHUNK
_NB = _CB // _BB

_NC = 2
_NS = 16
_NW = _NC * _NS


def _dist_body(z_ref, w_ref, z2_ref, w2_ref, idx_ref, loss_ref):
    wx = w_ref[...] + w_ref[...]
    w2b = w2_ref[...]
    part = jnp.zeros((1, 1), jnp.float32)
    for h in range(_NSUB):
        rows = pl.ds(h * _SB, _SB)
        zb = z_ref[rows, :]
        s2 = lax.dot_general(zb, wx, (((1,), (1,)), ((), ())))
        d2 = (z2_ref[rows, :] + w2b) - s2
        dists = jnp.sqrt(jnp.maximum(d2, 0.0))
        m = jnp.min(dists, axis=1, keepdims=True)
        colf = lax.broadcasted_iota(jnp.int32, dists.shape, 1).astype(jnp.float32)
        idx = jnp.min(jnp.where(dists == m, colf, 2048.0), axis=1)
        idx_ref[pl.ds(h * _SB, _SB)] = idx.astype(jnp.int32)
        part += jnp.sum(m * m, keepdims=True)
    @pl.when(pl.program_id(0) == 0)
    def _init():
        loss_ref[...] = jnp.zeros_like(loss_ref)
    loss_ref[...] += (1.0 + _BETA) * part


def _dist_call(z, W, z2, w2):
    return pl.pallas_call(
        _dist_body,
        grid=(_NB,),
        in_specs=[
            pl.BlockSpec((_BB, _E_DIM), lambda i: (i, 0)),
            pl.BlockSpec((_N_E, _E_DIM), lambda i: (0, 0)),
            pl.BlockSpec((_BB, 1), lambda i: (i, 0)),
            pl.BlockSpec((1, _N_E), lambda i: (0, 0)),
        ],
        out_specs=[
            pl.BlockSpec((_BB,), lambda i: (i,)),
            pl.BlockSpec((1, 1), lambda i: (0, 0)),
        ],
        out_shape=[
            jax.ShapeDtypeStruct((_CB,), jnp.int32),
            jax.ShapeDtypeStruct((1, 1), jnp.float32),
        ],
    )(z, W, z2, w2)


@functools.lru_cache(maxsize=1)
def _make_sc_gather():
    rows = _CB
    bpw = rows // _NW
    mesh = plsc.VectorSubcoreMesh(core_axis_name="c", subcore_axis_name="s")

    @functools.partial(
        pl.kernel,
        mesh=mesh,
        out_type=jax.ShapeDtypeStruct((rows, _E_DIM), jnp.float32),
        scratch_types=[
            pltpu.VMEM((bpw,), jnp.int32),
            pltpu.VMEM((bpw, _E_DIM), jnp.float32),
            pltpu.SemaphoreType.DMA,
        ],
    )
    def _sc_gather(w_hbm, idx_hbm, out_hbm, idx_v, rows_v, sem):
        wid = lax.axis_index("s") * _NC + lax.axis_index("c")
        base = wid * bpw
        pltpu.sync_copy(idx_hbm.at[pl.ds(base, bpw)], idx_v)
        pltpu.async_copy(w_hbm.at[idx_v], rows_v, sem).wait()
        pltpu.sync_copy(rows_v, out_hbm.at[pl.ds(base, bpw)])

    return _sc_gather


def kernel(z, W):
    z2 = jnp.sum(z * z, axis=1, keepdims=True)
    w2 = jnp.sum(W * W, axis=1)[None, :]
    gather = _make_sc_gather()
    zq_chunks = []
    loss = None
    for c in range(_NCHUNK):
        zc = lax.slice_in_dim(z, c * _CB, (c + 1) * _CB, axis=0)
        z2c = lax.slice_in_dim(z2, c * _CB, (c + 1) * _CB, axis=0)
        idx, lc = _dist_call(zc, W, z2c, w2)
        zq_chunks.append(gather(W, idx))
        loss = lc if loss is None else loss + lc
    z_q = zq_chunks[0] if _NCHUNK == 1 else jnp.concatenate(zq_chunks, axis=0)
    return (loss[0, 0], z_q)
